# baseline (speedup 1.0000x reference)
import dataclasses
import functools

import jax
import jax.numpy as jnp
from jax import lax
from jax.experimental import pallas as pl
from jax.experimental.pallas import tpu as pltpu
from jax.experimental.pallas import tpu_sc as plsc

V = 100000
D = 64
B = 16384
C = 20
K = 20

NC = 2
NS = 16
NW = NC * NS
L = 16

NB = 16
EPW = B // NW
NSTEPS = EPW // NB

IDX_CHUNK = 64
NCHUNK = NB * C // IDX_CHUNK
PAD = 32 - C


def _sc_body(emb, ctxi, tgti, negi, out1, out2p,
             ctx_idx0, neg_idx0, tgt_idx0, ctx_idx1, neg_idx1, tgt_idx1,
             ctx_rows0, neg_rows0, tgt_rows0, ctx_rows1, neg_rows1, tgt_rows1,
             pbuf, ybuf, fbuf, out1_buf,
             isem0, isem1, gsem0, gsem1):
  wid = lax.axis_index("s") * NC + lax.axis_index("c")
  iota = lax.iota(jnp.int32, L)
  colbase = iota * L

  idxb = [(ctx_idx0, neg_idx0, tgt_idx0), (ctx_idx1, neg_idx1, tgt_idx1)]
  rowb = [(ctx_rows0, neg_rows0, tgt_rows0), (ctx_rows1, neg_rows1, tgt_rows1)]
  isem = [isem0, isem1]
  gsem = [gsem0, gsem1]

  zero = jnp.zeros((L,), jnp.float32)
  for r in range(4 * 48):
    pbuf[pl.ds(r * L, L)] = zero
  lane_lt4 = iota < 4
  lane_mid = jnp.logical_and(iota >= 4, iota < 8)

  def colsum(buf, r0):
    base = colbase + r0 * L
    v = plsc.load_gather(buf, [base])
    for j in range(1, L):
      v = v + plsc.load_gather(buf, [base + j])
    return v

  def idx_descs(s, p):
    ibase = wid * (EPW * C) + s * (NB * C)
    tbase = wid * EPW + s * NB
    cv, nv, tv = idxb[p]
    return [
        pltpu.make_async_copy(ctxi.at[pl.ds(ibase, NB * C)], cv, isem[p]),
        pltpu.make_async_copy(negi.at[pl.ds(ibase, NB * C)], nv, isem[p]),
        pltpu.make_async_copy(tgti.at[pl.ds(tbase, NB)], tv, isem[p]),
    ]

  def gather_descs(p):
    cv, nv, tv = idxb[p]
    cr, nr, tr = rowb[p]
    descs = [pltpu.make_async_copy(emb.at[tv], tr, gsem[p])]
    for i in range(NCHUNK):
      sl = pl.ds(i * IDX_CHUNK, IDX_CHUNK)
      descs.append(pltpu.make_async_copy(emb.at[cv.at[sl]], cr.at[sl], gsem[p]))
      descs.append(pltpu.make_async_copy(emb.at[nv.at[sl]], nr.at[sl], gsem[p]))
    return descs

  def element(b, pb):
    cr, nr, tr = element.rows
    t = [tr[b, pl.ds(j * L, L)] for j in range(4)]
    rowbase = b * C
    a = [None] * 4
    for c in range(C):
      r = [cr[rowbase + c, pl.ds(j * L, L)] for j in range(4)]
      pv = (r[0] * t[0] + r[1] * t[1]) + (r[2] * t[2] + r[3] * t[3])
      pbuf[pl.ds((pb + c) * L, L)] = pv
      for j in range(4):
        a[j] = r[j] if a[j] is None else a[j] + r[j]
    for j in range(4):
      out1_buf[b, pl.ds(j * L, L)] = a[j]
    for k in range(K):
      r = [nr[rowbase + k, pl.ds(j * L, L)] for j in range(4)]
      pv = (r[0] * t[0] + r[1] * t[1]) + (r[2] * t[2] + r[3] * t[3])
      row = pb + (20 + (k - 16) if k >= 16 else 32 + k)
      pbuf[pl.ds(row * L, L)] = pv
    ea = jnp.exp(colsum(pbuf, pb))
    eb = jnp.exp(colsum(pbuf, pb + 16))
    ec = jnp.exp(colsum(pbuf, pb + 32))
    zf = jnp.zeros((L,), jnp.float32)
    ybuf[pl.ds(b * L, L)] = ea + jnp.where(lane_lt4, eb, zf)
    fbuf[pl.ds(b * L, L)] = ec + jnp.where(lane_mid, eb, zf)

  def compute(s, p, acc):
    element.rows = rowb[p]

    @pl.loop(0, NB, step=4)
    def _(b0):
      for u in range(4):
        element(b0 + u, u * 48)

    y_vec = colsum(ybuf, 0)
    f_vec = colsum(fbuf, 0)
    pltpu.sync_copy(out1_buf, out1.at[pl.ds(wid * EPW + s * NB, NB)])
    return acc + y_vec / f_vec

  for d in idx_descs(0, 0):
    d.start()
  for d in idx_descs(0, 0):
    d.wait()
  for d in gather_descs(0):
    d.start()
  for d in idx_descs(1, 1):
    d.start()

  def body(t, acc):
    for p in (0, 1):
      s = 2 * t + p

      @pl.when(s < NSTEPS - 1)
      def _():
        for d in idx_descs(0, 1 - p):
          d.wait()
        for d in gather_descs(1 - p):
          d.start()

      for d in gather_descs(p):
        d.wait()

      @pl.when(s < NSTEPS - 2)
      def _():
        for d in idx_descs(s + 2, p):
          d.start()

      acc = compute(s, p, acc)
    return acc

  acc = lax.fori_loop(0, NSTEPS // 2, body, zero)
  ybuf[pl.ds(0, L)] = acc
  pltpu.sync_copy(ybuf.at[pl.ds(0, L)], out2p.at[pl.ds(wid * L, L)])


@jax.jit
def _run(x_context, y_target, neg_samples, embedding):
  ctxi = x_context.reshape(B * C)
  negi = neg_samples.reshape(B * K)
  tgti = y_target.reshape(B)

  mesh = plsc.VectorSubcoreMesh(core_axis_name="c", subcore_axis_name="s")
  cp = pltpu.CompilerParams()
  if "needs_layout_passes" in pltpu.CompilerParams.__dataclass_fields__:
    cp = dataclasses.replace(cp, needs_layout_passes=False)
  if "use_tc_tiling_on_sc" in pltpu.CompilerParams.__dataclass_fields__:
    cp = dataclasses.replace(cp, use_tc_tiling_on_sc=False)
  sc = functools.partial(
      pl.kernel,
      compiler_params=cp,
      out_type=(
          jax.ShapeDtypeStruct((B, D), jnp.float32),
          jax.ShapeDtypeStruct((NW * L,), jnp.float32),
      ),
      mesh=mesh,
      scratch_types=(
          [pltpu.VMEM((NB * C,), jnp.int32),
           pltpu.VMEM((NB * K,), jnp.int32),
           pltpu.VMEM((NB,), jnp.int32)] * 2 +
          [pltpu.VMEM((NB * C, D), jnp.float32),
           pltpu.VMEM((NB * K, D), jnp.float32),
           pltpu.VMEM((NB, D), jnp.float32)] * 2 +
          [pltpu.VMEM((4 * 48 * L,), jnp.float32),
           pltpu.VMEM((NB * L,), jnp.float32),
           pltpu.VMEM((NB * L,), jnp.float32),
           pltpu.VMEM((NB, D), jnp.float32),
           pltpu.SemaphoreType.DMA,
           pltpu.SemaphoreType.DMA,
           pltpu.SemaphoreType.DMA,
           pltpu.SemaphoreType.DMA]
      ),
  )(_sc_body)
  out1, out2p = sc(embedding, ctxi, tgti, negi)
  return out1, jnp.sum(out2p)


def kernel(x_context, y_target, neg_samples, embedding):
  return _run(x_context, y_target, neg_samples, embedding)


# speedup vs baseline: 1.7827x; 1.7827x over previous
import dataclasses
import functools

import jax
import jax.numpy as jnp
from jax import lax
from jax.experimental import pallas as pl
from jax.experimental.pallas import tpu as pltpu
from jax.experimental.pallas import tpu_sc as plsc

V = 100000
D = 64
B = 16384
C = 20
K = 20

NC = 2
NS = 16
NW = NC * NS
L = 16

NB = 16
EPW = B // NW
NSTEPS = EPW // NB

IDX_CHUNK = 64
NCHUNK = NB * C // IDX_CHUNK
PAD = 32 - C


def _sc_body(emb, ctxi, tgti, negi, out1, out2p,
             ctx_idx0, neg_idx0, tgt_idx0, ctx_idx1, neg_idx1, tgt_idx1,
             ctx_rows0, neg_rows0, tgt_rows0, ctx_rows1, neg_rows1, tgt_rows1,
             pbuf, ybuf, fbuf, out1_buf,
             isem0, isem1, gsem0, gsem1):
  wid = lax.axis_index("s") * NC + lax.axis_index("c")
  iota = lax.iota(jnp.int32, L)
  colbase = iota * L

  idxb = [(ctx_idx0, neg_idx0, tgt_idx0), (ctx_idx1, neg_idx1, tgt_idx1)]
  rowb = [(ctx_rows0, neg_rows0, tgt_rows0), (ctx_rows1, neg_rows1, tgt_rows1)]
  isem = [isem0, isem1]
  gsem = [gsem0, gsem1]

  zero = jnp.zeros((L,), jnp.float32)
  for r in range(4 * 48):
    pbuf[pl.ds(r * L, L)] = zero
  lane_lt4 = iota < 4
  lane_mid = jnp.logical_and(iota >= 4, iota < 8)

  def colsum(buf, r0):
    base = colbase + r0 * L
    v = plsc.load_gather(buf, [base])
    for j in range(1, L):
      v = v + plsc.load_gather(buf, [base + j])
    return v

  def idx_descs(s, p):
    ibase = wid * (EPW * C) + s * (NB * C)
    tbase = wid * EPW + s * NB
    cv, nv, tv = idxb[p]
    return [
        pltpu.make_async_copy(ctxi.at[pl.ds(ibase, NB * C)], cv, isem[p]),
        pltpu.make_async_copy(negi.at[pl.ds(ibase, NB * C)], nv, isem[p]),
        pltpu.make_async_copy(tgti.at[pl.ds(tbase, NB)], tv, isem[p]),
    ]

  def gather_descs(p):
    cv, nv, tv = idxb[p]
    cr, nr, tr = rowb[p]
    descs = [pltpu.make_async_copy(emb.at[tv], tr, gsem[p])]
    for i in range(NCHUNK):
      sl = pl.ds(i * IDX_CHUNK, IDX_CHUNK)
      descs.append(pltpu.make_async_copy(emb.at[cv.at[sl]], cr.at[sl], gsem[p]))
      descs.append(pltpu.make_async_copy(emb.at[nv.at[sl]], nr.at[sl], gsem[p]))
    return descs

  def element(b, pb):
    cr, nr, tr = element.rows
    t = [tr[b, pl.ds(j * L, L)] for j in range(4)]
    rowbase = b * C
    a = [None] * 4
    for c in range(C):
      r = [cr[rowbase + c, pl.ds(j * L, L)] for j in range(4)]
      pv = (r[0] * t[0] + r[1] * t[1]) + (r[2] * t[2] + r[3] * t[3])
      pbuf[pl.ds((pb + c) * L, L)] = pv
      for j in range(4):
        a[j] = r[j] if a[j] is None else a[j] + r[j]
    for j in range(4):
      out1_buf[b, pl.ds(j * L, L)] = a[j]
    for k in range(K):
      r = [nr[rowbase + k, pl.ds(j * L, L)] for j in range(4)]
      pv = (r[0] * t[0] + r[1] * t[1]) + (r[2] * t[2] + r[3] * t[3])
      row = pb + (20 + (k - 16) if k >= 16 else 32 + k)
      pbuf[pl.ds(row * L, L)] = pv
    ea = jnp.exp(colsum(pbuf, pb))
    eb = jnp.exp(colsum(pbuf, pb + 16))
    ec = jnp.exp(colsum(pbuf, pb + 32))
    zf = jnp.zeros((L,), jnp.float32)
    ybuf[pl.ds(b * L, L)] = ea + jnp.where(lane_lt4, eb, zf)
    fbuf[pl.ds(b * L, L)] = ec + jnp.where(lane_mid, eb, zf)

  def compute(s, p, acc):
    element.rows = rowb[p]

    @pl.loop(0, NB)
    def _(b):
      element(b, (b & 1) * 48)

    y_vec = colsum(ybuf, 0)
    f_vec = colsum(fbuf, 0)
    pltpu.sync_copy(out1_buf, out1.at[pl.ds(wid * EPW + s * NB, NB)])
    return acc + y_vec / f_vec

  for d in idx_descs(0, 0):
    d.start()
  for d in idx_descs(0, 0):
    d.wait()
  for d in gather_descs(0):
    d.start()
  for d in idx_descs(1, 1):
    d.start()

  def body(t, acc):
    for p in (0, 1):
      s = 2 * t + p

      @pl.when(s < NSTEPS - 1)
      def _():
        for d in idx_descs(0, 1 - p):
          d.wait()
        for d in gather_descs(1 - p):
          d.start()

      for d in gather_descs(p):
        d.wait()

      @pl.when(s < NSTEPS - 2)
      def _():
        for d in idx_descs(s + 2, p):
          d.start()

      acc = compute(s, p, acc)
    return acc

  acc = lax.fori_loop(0, NSTEPS // 2, body, zero)
  ybuf[pl.ds(0, L)] = acc
  pltpu.sync_copy(ybuf.at[pl.ds(0, L)], out2p.at[pl.ds(wid * L, L)])


@jax.jit
def _run(x_context, y_target, neg_samples, embedding):
  ctxi = x_context.reshape(B * C)
  negi = neg_samples.reshape(B * K)
  tgti = y_target.reshape(B)

  mesh = plsc.VectorSubcoreMesh(core_axis_name="c", subcore_axis_name="s")
  cp = pltpu.CompilerParams()
  if "needs_layout_passes" in pltpu.CompilerParams.__dataclass_fields__:
    cp = dataclasses.replace(cp, needs_layout_passes=False)
  if "use_tc_tiling_on_sc" in pltpu.CompilerParams.__dataclass_fields__:
    cp = dataclasses.replace(cp, use_tc_tiling_on_sc=False)
  sc = functools.partial(
      pl.kernel,
      compiler_params=cp,
      out_type=(
          jax.ShapeDtypeStruct((B, D), jnp.float32),
          jax.ShapeDtypeStruct((NW * L,), jnp.float32),
      ),
      mesh=mesh,
      scratch_types=(
          [pltpu.VMEM((NB * C,), jnp.int32),
           pltpu.VMEM((NB * K,), jnp.int32),
           pltpu.VMEM((NB,), jnp.int32)] * 2 +
          [pltpu.VMEM((NB * C, D), jnp.float32),
           pltpu.VMEM((NB * K, D), jnp.float32),
           pltpu.VMEM((NB, D), jnp.float32)] * 2 +
          [pltpu.VMEM((4 * 48 * L,), jnp.float32),
           pltpu.VMEM((NB * L,), jnp.float32),
           pltpu.VMEM((NB * L,), jnp.float32),
           pltpu.VMEM((NB, D), jnp.float32),
           pltpu.SemaphoreType.DMA,
           pltpu.SemaphoreType.DMA,
           pltpu.SemaphoreType.DMA,
           pltpu.SemaphoreType.DMA]
      ),
  )(_sc_body)
  out1, out2p = sc(embedding, ctxi, tgti, negi)
  return out1, jnp.sum(out2p)


def kernel(x_context, y_target, neg_samples, embedding):
  return _run(x_context, y_target, neg_samples, embedding)


# speedup vs baseline: 2.7926x; 1.5665x over previous
import dataclasses
import functools

import jax
import jax.numpy as jnp
from jax import lax
from jax.experimental import pallas as pl
from jax.experimental.pallas import tpu as pltpu
from jax.experimental.pallas import tpu_sc as plsc

V = 100000
D = 64
B = 16384
C = 20
K = 20

NC = 2
NS = 16
NW = NC * NS
L = 16

NB = 16
EPW = B // NW
NSTEPS = EPW // NB

IDX_CHUNK = 64
NCHUNK = NB * C // IDX_CHUNK
PAD = 32 - C


def _sc_body(emb, ctxi, tgti, negi, out1, out2p,
             ctx_idx0, neg_idx0, tgt_idx0, ctx_idx1, neg_idx1, tgt_idx1,
             ctx_rows0, neg_rows0, tgt_rows0, ctx_rows1, neg_rows1, tgt_rows1,
             ybuf, fbuf, out1_buf,
             isem0, isem1, gsem0, gsem1):
  wid = lax.axis_index("s") * NC + lax.axis_index("c")
  iota = lax.iota(jnp.int32, L)
  colbase = iota * L

  idxb = [(ctx_idx0, neg_idx0, tgt_idx0), (ctx_idx1, neg_idx1, tgt_idx1)]
  rowb = [(ctx_rows0, neg_rows0, tgt_rows0), (ctx_rows1, neg_rows1, tgt_rows1)]
  isem = [isem0, isem1]
  gsem = [gsem0, gsem1]

  zero = jnp.zeros((L,), jnp.float32)

  def colsum(buf, r0):
    base = colbase + r0 * L
    v = plsc.load_gather(buf, [base])
    for j in range(1, L):
      v = v + plsc.load_gather(buf, [base + j])
    return v

  def idx_descs(s, p):
    ibase = wid * (EPW * C) + s * (NB * C)
    tbase = wid * EPW + s * NB
    cv, nv, tv = idxb[p]
    return [
        pltpu.make_async_copy(ctxi.at[pl.ds(ibase, NB * C)], cv, isem[p]),
        pltpu.make_async_copy(negi.at[pl.ds(ibase, NB * C)], nv, isem[p]),
        pltpu.make_async_copy(tgti.at[pl.ds(tbase, NB)], tv, isem[p]),
    ]

  def gather_descs(p):
    cv, nv, tv = idxb[p]
    cr, nr, tr = rowb[p]
    descs = [pltpu.make_async_copy(emb.at[tv], tr, gsem[p])]
    for i in range(NCHUNK):
      sl = pl.ds(i * IDX_CHUNK, IDX_CHUNK)
      descs.append(pltpu.make_async_copy(emb.at[cv.at[sl]], cr.at[sl], gsem[p]))
      descs.append(pltpu.make_async_copy(emb.at[nv.at[sl]], nr.at[sl], gsem[p]))
    return descs

  perm = {k: (iota ^ k).reshape(L, 1) for k in (8, 4, 2, 1)}
  mask = {k: (iota & k) == 0 for k in (8, 4, 2, 1)}
  lane4 = (iota & 3) == 0
  zf = jnp.zeros((L,), jnp.float32)
  gdn = lax.GatherDimensionNumbers(
      offset_dims=(), collapsed_slice_dims=(0,), start_index_map=(0,))

  def shuf(v, k):
    return v + lax.gather(v, perm[k], gdn, slice_sizes=(1,),
                          mode=lax.GatherScatterMode.PROMISE_IN_BOUNDS)

  def comb(x, y, k):
    return jnp.where(mask[k], shuf(x, k), shuf(y, k))

  def red20(ps):
    cur = ps[:16]
    for k in (8, 4, 2, 1):
      cur = [comb(cur[2 * i], cur[2 * i + 1], k) for i in range(len(cur) // 2)]
    u = comb(ps[16], ps[17], 8)
    v = comb(ps[18], ps[19], 8)
    for k in (4, 2, 1):
      u, v = shuf(u, k), shuf(v, k)
    w = jnp.where(mask[4], u, v)
    return jnp.exp(cur[0]), jnp.where(lane4, jnp.exp(w), zf)

  def element(b):
    cr, nr, tr = element.rows
    t = [tr[b, pl.ds(j * L, L)] for j in range(4)]
    rowbase = b * C
    cps, nps = [], []
    for c in range(C):
      r = [cr[rowbase + c, pl.ds(j * L, L)] for j in range(4)]
      cps.append((r[0] * t[0] + r[1] * t[1]) + (r[2] * t[2] + r[3] * t[3]))
      for j in range(4):
        if c == 0:
          out1_buf[b, pl.ds(j * L, L)] = r[j]
        else:
          plsc.addupdate(out1_buf.at[b, pl.ds(j * L, L)], r[j])
    for k in range(K):
      r = [nr[rowbase + k, pl.ds(j * L, L)] for j in range(4)]
      nps.append((r[0] * t[0] + r[1] * t[1]) + (r[2] * t[2] + r[3] * t[3]))
    ch, ct = red20(cps)
    nh, nt = red20(nps)
    ybuf[pl.ds(b * L, L)] = ch + ct
    fbuf[pl.ds(b * L, L)] = nh + nt

  def compute(s, p, acc):
    element.rows = rowb[p]

    @pl.loop(0, NB)
    def _(b):
      element(b)

    y_vec = colsum(ybuf, 0)
    f_vec = colsum(fbuf, 0)
    pltpu.sync_copy(out1_buf, out1.at[pl.ds(wid * EPW + s * NB, NB)])
    return acc + y_vec / f_vec

  for d in idx_descs(0, 0):
    d.start()
  for d in idx_descs(0, 0):
    d.wait()
  for d in gather_descs(0):
    d.start()
  for d in idx_descs(1, 1):
    d.start()

  def body(t, acc):
    for p in (0, 1):
      s = 2 * t + p

      @pl.when(s < NSTEPS - 1)
      def _():
        for d in idx_descs(0, 1 - p):
          d.wait()
        for d in gather_descs(1 - p):
          d.start()

      for d in gather_descs(p):
        d.wait()

      @pl.when(s < NSTEPS - 2)
      def _():
        for d in idx_descs(s + 2, p):
          d.start()

      acc = compute(s, p, acc)
    return acc

  acc = lax.fori_loop(0, NSTEPS // 2, body, zero)
  ybuf[pl.ds(0, L)] = acc
  pltpu.sync_copy(ybuf.at[pl.ds(0, L)], out2p.at[pl.ds(wid * L, L)])


@jax.jit
def _run(x_context, y_target, neg_samples, embedding):
  ctxi = x_context.reshape(B * C)
  negi = neg_samples.reshape(B * K)
  tgti = y_target.reshape(B)

  mesh = plsc.VectorSubcoreMesh(core_axis_name="c", subcore_axis_name="s")
  cp = pltpu.CompilerParams()
  if "needs_layout_passes" in pltpu.CompilerParams.__dataclass_fields__:
    cp = dataclasses.replace(cp, needs_layout_passes=False)
  if "use_tc_tiling_on_sc" in pltpu.CompilerParams.__dataclass_fields__:
    cp = dataclasses.replace(cp, use_tc_tiling_on_sc=False)
  sc = functools.partial(
      pl.kernel,
      compiler_params=cp,
      out_type=(
          jax.ShapeDtypeStruct((B, D), jnp.float32),
          jax.ShapeDtypeStruct((NW * L,), jnp.float32),
      ),
      mesh=mesh,
      scratch_types=(
          [pltpu.VMEM((NB * C,), jnp.int32),
           pltpu.VMEM((NB * K,), jnp.int32),
           pltpu.VMEM((NB,), jnp.int32)] * 2 +
          [pltpu.VMEM((NB * C, D), jnp.float32),
           pltpu.VMEM((NB * K, D), jnp.float32),
           pltpu.VMEM((NB, D), jnp.float32)] * 2 +
          [pltpu.VMEM((NB * L,), jnp.float32),
           pltpu.VMEM((NB * L,), jnp.float32),
           pltpu.VMEM((NB, D), jnp.float32),
           pltpu.SemaphoreType.DMA,
           pltpu.SemaphoreType.DMA,
           pltpu.SemaphoreType.DMA,
           pltpu.SemaphoreType.DMA]
      ),
  )(_sc_body)
  out1, out2p = sc(embedding, ctxi, tgti, negi)
  return out1, jnp.sum(out2p)


def kernel(x_context, y_target, neg_samples, embedding):
  return _run(x_context, y_target, neg_samples, embedding)
